# Initial kernel scaffold; baseline (speedup 1.0000x reference)
import jax
import jax.numpy as jnp
from jax.experimental import pallas as pl


def kernel(end_points, start_points, W0, b0, W1, b1, W2, b2):
    raise NotImplementedError("write your pallas kernel here")


# speedup vs baseline: 4.8995x; 4.8995x over previous
import functools

import jax
import jax.numpy as jnp
from jax.experimental import pallas as pl

MK = 64
EPS = 1e-6


def _tc_a_body(end_ref, start_ref, w0t_ref, b0_ref, a0_ref, be0_ref, t1c_ref):
    d = end_ref[...] - start_ref[...]
    w0t = w0t_ref[...]
    a0 = jnp.dot(d, w0t, preferred_element_type=jnp.float32)
    be0 = jnp.dot(start_ref[...], w0t, preferred_element_type=jnp.float32) + b0_ref[...]
    zr = a0 + be0
    denom = zr - be0
    dval = jnp.abs(denom) > EPS
    safe = jnp.where(dval, denom, jnp.ones_like(denom))
    a1 = jnp.negative(be0) / safe
    v1 = dval & (a1 > EPS) & (a1 < 1.0 - EPS)
    a0_ref[...] = a0
    be0_ref[...] = be0
    t1c_ref[...] = jnp.where(v1, a1, jnp.full_like(a1, 2.0))


def _tc_a(end_points, start_points, w0t, b0):
    B, H = end_points.shape[0], w0t.shape[1]
    return pl.pallas_call(
        _tc_a_body,
        out_shape=[
            jax.ShapeDtypeStruct((B, H), jnp.float32),
            jax.ShapeDtypeStruct((B, H), jnp.float32),
            jax.ShapeDtypeStruct((B, H), jnp.float32),
        ],
    )(end_points, start_points, w0t, b0.reshape(1, H))


def _tc_b_body(a0_ref, be0_ref, t1_ref, n1_ref, k1_ref, w1t_ref, b1_ref,
               z2_ref, t2c_ref, a2_ref):
    B, K = t1_ref.shape
    H = a0_ref.shape[1]
    a0 = a0_ref[...]
    be0 = be0_ref[...]
    t1 = t1_ref[...]
    n1 = n1_ref[...]
    k1 = k1_ref[...]
    zr = a0 + be0
    denom = zr - be0
    slot = jax.lax.broadcasted_iota(jnp.int32, (B, K), 1)
    is_k1 = (slot == k1)
    P = is_k1.astype(jnp.float32)
    R = jnp.where(is_k1, 0.0, t1)
    z1 = (P[:, :, None] * a0[:, None, :] + be0[:, None, :]
          + R[:, :, None] * denom[:, None, :])
    M = (slot < n1).astype(jnp.float32)
    h1 = jnp.maximum(z1, 0.0) * M[:, :, None]
    z2 = (jnp.dot(h1.reshape(B * K, H), w1t_ref[...],
                  preferred_element_type=jnp.float32)
          + b1_ref[...]).reshape(B, K, H)
    z2_ref[...] = z2
    tl = t1[:, :-1]
    dt = t1[:, 1:] - tl
    zl = z2[:, :-1, :]
    den2 = z2[:, 1:, :] - zl
    dval = jnp.abs(den2) > EPS
    safe = jnp.where(dval, den2, jnp.ones_like(den2))
    a2 = jnp.negative(zl) / safe
    v2 = dval & (a2 > EPS) & (a2 < 1.0 - EPS)
    t2c3 = jnp.where(v2, tl[:, :, None] + a2 * dt[:, :, None],
                     jnp.full_like(a2, 2.0))
    segf = (jax.lax.broadcasted_iota(jnp.int32, (B, K - 1), 1)
            < (n1 - 1)).astype(jnp.float32)
    s3 = segf[:, :, None]
    t2c_ref[...] = t2c3 * s3 + (1.0 - s3) * 2.0
    a2_ref[...] = a2


def _tc_b(a0, be0, t1, n1, k1pos, w1t, b1):
    B, K = t1.shape
    H = a0.shape[1]
    return pl.pallas_call(
        _tc_b_body,
        out_shape=[
            jax.ShapeDtypeStruct((B, K, H), jnp.float32),
            jax.ShapeDtypeStruct((B, K - 1, H), jnp.float32),
            jax.ShapeDtypeStruct((B, K - 1, H), jnp.float32),
        ],
    )(a0, be0, t1, n1.reshape(B, 1), k1pos.reshape(B, 1), w1t, b1.reshape(1, H))


def _tc_c_body(h2_ref, w2t_ref, b2_ref, n2_ref, y_ref, v2_ref):
    B, K, H = h2_ref.shape
    O = w2t_ref.shape[1]
    y = (jnp.dot(h2_ref[...].reshape(B * K, H), w2t_ref[...],
                 preferred_element_type=jnp.float32)
         + b2_ref[...]).reshape(B, K, O)
    slot = jax.lax.broadcasted_iota(jnp.int32, (B, K), 1)
    v = slot < n2_ref[...]
    vf = v.astype(jnp.float32)
    y_ref[...] = y * vf[:, :, None]
    v2_ref[...] = v.astype(jnp.int32)


def _tc_c(h2, w2t, b2, n2):
    B, K, H = h2.shape
    O = w2t.shape[1]
    return pl.pallas_call(
        _tc_c_body,
        out_shape=[
            jax.ShapeDtypeStruct((B, K, O), jnp.float32),
            jax.ShapeDtypeStruct((B, K), jnp.int32),
        ],
    )(h2, w2t, b2.reshape(1, O), n2.reshape(B, 1))


def _layer1_build(t1c):
    B = t1c.shape[0]
    s = jnp.sort(t1c, axis=1)
    c = jnp.sum(t1c < 1.5, axis=1).astype(jnp.int32)
    slot = jax.lax.broadcasted_iota(jnp.int32, (B, MK), 1)
    shifted = jnp.concatenate([jnp.zeros((B, 1), jnp.float32), s[:, :-1]], axis=1)
    t1 = jnp.where(slot == 0, 0.0, jnp.where(slot - 1 < c[:, None], shifted, 1.0))
    n1 = jnp.minimum(2 + c, MK)
    k1pos = jnp.where(c <= MK - 2, 1 + c, -1)
    return t1, n1, k1pos


def _layer2_build(t1, n1, t2c, a2, z2):
    B, K = t1.shape
    S = K - 1
    idx = jnp.argsort(t2c, axis=2)
    ts = jnp.take_along_axis(t2c, idx, axis=2)
    asrt = jnp.take_along_axis(a2, idx, axis=2)
    seg = jax.lax.broadcasted_iota(jnp.int32, (B, S), 1)
    old_v = seg < n1[:, None]
    t_seg = jnp.concatenate([t1[:, :S, None], ts], axis=2)
    a_seg = jnp.concatenate([jnp.zeros((B, S, 1), jnp.float32), asrt], axis=2)
    v_seg = jnp.concatenate([old_v[:, :, None], ts < 1.5], axis=2)
    s_seg = jnp.broadcast_to(seg[:, :, None], (B, S, K + 1))
    L = S * (K + 1)
    t_all = jnp.concatenate([t_seg.reshape(B, L), t1[:, -1:]], axis=1)
    a_all = jnp.concatenate([a_seg.reshape(B, L), jnp.zeros((B, 1), jnp.float32)], axis=1)
    v_all = jnp.concatenate([v_seg.reshape(B, L), (n1 == K)[:, None]], axis=1)
    s_all = jnp.concatenate([s_seg.reshape(B, L), jnp.full((B, 1), S, jnp.int32)], axis=1)
    pos = jnp.cumsum(v_all.astype(jnp.int32), axis=1) - 1
    keep = v_all & (pos < MK)
    idx_out = jnp.where(keep, pos, 0)
    b_idx = jnp.arange(B)[:, None]
    t2 = jnp.zeros((B, MK), jnp.float32).at[b_idx, idx_out].add(jnp.where(keep, t_all, 0.0))
    a2s = jnp.zeros((B, MK), jnp.float32).at[b_idx, idx_out].add(jnp.where(keep, a_all, 0.0))
    s2 = jnp.zeros((B, MK), jnp.int32).at[b_idx, idx_out].add(jnp.where(keep, s_all, 0))
    n2 = jnp.minimum(jnp.sum(v_all.astype(jnp.int32), axis=1), MK)
    slot = jax.lax.broadcasted_iota(jnp.int32, (B, MK), 1)
    t2 = jnp.where(slot < n2[:, None], t2, 1.0)
    zl = jnp.take_along_axis(z2, s2[:, :, None], axis=1)
    zr2 = jnp.take_along_axis(z2, jnp.minimum(s2 + 1, K - 1)[:, :, None], axis=1)
    zk = zl + a2s[:, :, None] * (zr2 - zl)
    h2 = jnp.where(slot[:, :, None] < n2[:, None, None], jnp.maximum(zk, 0.0), 0.0)
    return t2, n2, h2


def kernel(end_points, start_points, W0, b0, W1, b1, W2, b2):
    a0, be0, t1c = _tc_a(end_points, start_points, W0.T, b0)
    t1, n1, k1pos = _layer1_build(t1c)
    z2, t2c, a2 = _tc_b(a0, be0, t1, n1, k1pos, W1.T, b1)
    t2, n2, h2 = _layer2_build(t1, n1, t2c, a2, z2)
    y, v2 = _tc_c(h2, W2.T, b2, n2)
    return t2, v2.astype(bool), y


# speedup vs baseline: 56.9412x; 11.6219x over previous
import functools

import jax
import jax.numpy as jnp
from jax import lax
from jax.experimental import pallas as pl
from jax.experimental.pallas import tpu as pltpu
from jax.experimental.pallas import tpu_sc as plsc

MK = 64
EPS = 1e-6

_NC, _NS, _NL = 2, 16, 16
_NW = _NC * _NS
_RPW = 128 // _NW


def _rev(x):
    return lax.rev(x, (0,))


def _merge2k(a, b):
    br = _rev(b)
    lo = jnp.minimum(a, br)
    hi = jnp.maximum(a, br)
    return lax.sort(lo, dimension=0), lax.sort(hi, dimension=0)


def _merge4k(a0, a1, b0, b1):
    r0 = _rev(b1)
    r1 = _rev(b0)
    lo0, lo1 = jnp.minimum(a0, r0), jnp.minimum(a1, r1)
    hi0, hi1 = jnp.maximum(a0, r0), jnp.maximum(a1, r1)

    def fin(x0, x1):
        n0 = jnp.minimum(x0, x1)
        n1 = jnp.maximum(x0, x1)
        return lax.sort(n0, dimension=0), lax.sort(n1, dimension=0)

    o0, o1 = fin(lo0, lo1)
    o2, o3 = fin(hi0, hi1)
    return o0, o1, o2, o3


def _merge2kv(ka, va, kb, vb):
    kbr, vbr = _rev(kb), _rev(vb)
    m = ka <= kbr
    klo = jnp.where(m, ka, kbr)
    vlo = jnp.where(m, va, vbr)
    khi = jnp.where(m, kbr, ka)
    vhi = jnp.where(m, vbr, va)
    s0 = plsc.sort_key_val(klo, vlo)
    s1 = plsc.sort_key_val(khi, vhi)
    return s0[0], s0[1], s1[0], s1[1]


def _merge4kv(ka0, va0, ka1, va1, kb0, vb0, kb1, vb1):
    kr0, vr0 = _rev(kb1), _rev(vb1)
    kr1, vr1 = _rev(kb0), _rev(vb0)
    m0 = ka0 <= kr0
    m1 = ka1 <= kr1
    kl0 = jnp.where(m0, ka0, kr0)
    vl0 = jnp.where(m0, va0, vr0)
    kl1 = jnp.where(m1, ka1, kr1)
    vl1 = jnp.where(m1, va1, vr1)
    kh0 = jnp.where(m0, kr0, ka0)
    vh0 = jnp.where(m0, vr0, va0)
    kh1 = jnp.where(m1, kr1, ka1)
    vh1 = jnp.where(m1, vr1, va1)

    def fin(x0, xv0, x1, xv1):
        m = x0 <= x1
        n0 = jnp.where(m, x0, x1)
        nv0 = jnp.where(m, xv0, xv1)
        n1 = jnp.where(m, x1, x0)
        nv1 = jnp.where(m, xv1, xv0)
        s0 = plsc.sort_key_val(n0, nv0)
        s1 = plsc.sort_key_val(n1, nv1)
        return s0[0], s0[1], s1[0], s1[1]

    o0, ov0, o1, ov1 = fin(kl0, vl0, kl1, vl1)
    o2, ov2, o3, ov3 = fin(kh0, vh0, kh1, vh1)
    return (o0, ov0, o1, ov1, o2, ov2, o3, ov3)


def _tc_a_body(end_ref, start_ref, w0t_ref, b0_ref, a0_ref, be0_ref, t1c_ref):
    d = end_ref[...] - start_ref[...]
    w0t = w0t_ref[...]
    a0 = jnp.dot(d, w0t, preferred_element_type=jnp.float32)
    be0 = jnp.dot(start_ref[...], w0t, preferred_element_type=jnp.float32) + b0_ref[...]
    zr = a0 + be0
    denom = zr - be0
    dval = jnp.abs(denom) > EPS
    safe = jnp.where(dval, denom, jnp.ones_like(denom))
    a1 = jnp.negative(be0) / safe
    v1 = dval & (a1 > EPS) & (a1 < 1.0 - EPS)
    a0_ref[...] = a0
    be0_ref[...] = be0
    t1c_ref[...] = jnp.where(v1, a1, jnp.full_like(a1, 2.0))


def _tc_a(end_points, start_points, w0t, b0):
    B, H = end_points.shape[0], w0t.shape[1]
    return pl.pallas_call(
        _tc_a_body,
        out_shape=[
            jax.ShapeDtypeStruct((B, H), jnp.float32),
            jax.ShapeDtypeStruct((B, H), jnp.float32),
            jax.ShapeDtypeStruct((B, H), jnp.float32),
        ],
    )(end_points, start_points, w0t, b0.reshape(1, H))


def _tc_b_body(a0_ref, be0_ref, t1_ref, n1_ref, k1_ref, w1t_ref, b1_ref,
               z2_ref, t2c_ref, a2_ref):
    B, K = t1_ref.shape
    H = a0_ref.shape[1]
    a0 = a0_ref[...]
    be0 = be0_ref[...]
    t1 = t1_ref[...]
    n1 = n1_ref[...]
    k1 = k1_ref[...]
    zr = a0 + be0
    denom = zr - be0
    slot = jax.lax.broadcasted_iota(jnp.int32, (B, K), 1)
    is_k1 = (slot == k1)
    P = is_k1.astype(jnp.float32)
    R = jnp.where(is_k1, 0.0, t1)
    z1 = (P[:, :, None] * a0[:, None, :] + be0[:, None, :]
          + R[:, :, None] * denom[:, None, :])
    M = (slot < n1).astype(jnp.float32)
    h1 = jnp.maximum(z1, 0.0) * M[:, :, None]
    z2 = (jnp.dot(h1.reshape(B * K, H), w1t_ref[...],
                  preferred_element_type=jnp.float32)
          + b1_ref[...]).reshape(B, K, H)
    z2_ref[...] = z2
    tl = t1[:, :-1]
    dt = t1[:, 1:] - tl
    zl = z2[:, :-1, :]
    den2 = z2[:, 1:, :] - zl
    dval = jnp.abs(den2) > EPS
    safe = jnp.where(dval, den2, jnp.ones_like(den2))
    a2 = jnp.negative(zl) / safe
    v2 = dval & (a2 > EPS) & (a2 < 1.0 - EPS)
    t2c3 = jnp.where(v2, tl[:, :, None] + a2 * dt[:, :, None],
                     jnp.full_like(a2, 2.0))
    segf = (jax.lax.broadcasted_iota(jnp.int32, (B, K - 1), 1)
            < (n1 - 1)).astype(jnp.float32)
    s3 = segf[:, :, None]
    t2c_ref[...] = t2c3 * s3 + (1.0 - s3) * 2.0
    a2_ref[...] = a2


def _tc_b(a0, be0, t1, n1, k1pos, w1t, b1):
    B, K = t1.shape
    H = a0.shape[1]
    return pl.pallas_call(
        _tc_b_body,
        out_shape=[
            jax.ShapeDtypeStruct((B, K, H), jnp.float32),
            jax.ShapeDtypeStruct((B, K - 1, H), jnp.float32),
            jax.ShapeDtypeStruct((B, K - 1, H), jnp.float32),
        ],
    )(a0, be0, t1, n1.reshape(B, 1), k1pos.reshape(B, 1), w1t, b1.reshape(1, H))


def _tc_c_body(h2_ref, w2t_ref, b2_ref, n2_ref, y_ref, v2_ref):
    B, K, H = h2_ref.shape
    O = w2t_ref.shape[1]
    y = (jnp.dot(h2_ref[...].reshape(B * K, H), w2t_ref[...],
                 preferred_element_type=jnp.float32)
         + b2_ref[...]).reshape(B, K, O)
    slot = jax.lax.broadcasted_iota(jnp.int32, (B, K), 1)
    v = slot < n2_ref[...]
    vf = v.astype(jnp.float32)
    y_ref[...] = y * vf[:, :, None]
    v2_ref[...] = v.astype(jnp.int32)


def _tc_c(h2, w2t, b2, n2):
    B, K, H = h2.shape
    O = w2t.shape[1]
    return pl.pallas_call(
        _tc_c_body,
        out_shape=[
            jax.ShapeDtypeStruct((B, K, O), jnp.float32),
            jax.ShapeDtypeStruct((B, K), jnp.int32),
        ],
    )(h2, w2t, b2.reshape(1, O), n2.reshape(B, 1))


def _sc1_body(t1c_hbm, t1_hbm, meta_hbm, in_v, out_v, meta_v):
    wid = lax.axis_index("c") * _NS + lax.axis_index("s")
    iota = lax.broadcasted_iota(jnp.int32, (_NL,), 0)
    one = jnp.full((_NL,), 1.0, jnp.float32)
    for r in range(_RPW):
        ray = wid * _RPW + r
        pltpu.sync_copy(t1c_hbm.at[ray], in_v)
        ks = []
        total = jnp.int32(0)
        for c in range(4):
            x = in_v[pl.ds(c * _NL, _NL)]
            total = total + jnp.sum((x < 1.5).astype(jnp.int32))
            ks.append(lax.sort(x, dimension=0))
        a0, a1 = _merge2k(ks[0], ks[1])
        b0, b1 = _merge2k(ks[2], ks[3])
        s4 = _merge4k(a0, a1, b0, b1)
        out_v[pl.ds(0, _NL)] = jnp.where(iota == 0, 0.0, one)
        for c in range(1, 4):
            out_v[pl.ds(c * _NL, _NL)] = one
        lim = jnp.minimum(total, MK - 1)
        for c in range(4):
            pos = iota + (c * _NL)
            plsc.store_scatter(out_v, [pos + 1], s4[c], mask=pos < lim)
        n1s = jnp.minimum(total + 2, MK)
        k1s = jnp.where(total <= MK - 2, total + 1, -1)
        meta_v[pl.ds(0, _NL)] = jnp.where(iota == 0, n1s,
                                          jnp.where(iota == 1, k1s, 0))
        pltpu.sync_copy(out_v, t1_hbm.at[ray])
        pltpu.sync_copy(meta_v, meta_hbm.at[ray])


def _sc1(t1c):
    B = t1c.shape[0]
    mesh = plsc.VectorSubcoreMesh(core_axis_name="c", subcore_axis_name="s")
    f = functools.partial(
        pl.kernel,
        out_type=[
            jax.ShapeDtypeStruct((B, MK), jnp.float32),
            jax.ShapeDtypeStruct((B, 16), jnp.int32),
        ],
        mesh=mesh,
        scratch_types=[
            pltpu.VMEM((MK,), jnp.float32),
            pltpu.VMEM((MK,), jnp.float32),
            pltpu.VMEM((16,), jnp.int32),
        ],
        compiler_params=pltpu.CompilerParams(needs_layout_passes=False),
    )(_sc1_body)
    t1, meta = f(t1c)
    return t1, meta


def _layer2_build(t1, n1, t2c, a2, z2):
    B, K = t1.shape
    S = K - 1
    idx = jnp.argsort(t2c, axis=2)
    ts = jnp.take_along_axis(t2c, idx, axis=2)
    asrt = jnp.take_along_axis(a2, idx, axis=2)
    seg = jax.lax.broadcasted_iota(jnp.int32, (B, S), 1)
    old_v = seg < n1
    t_seg = jnp.concatenate([t1[:, :S, None], ts], axis=2)
    a_seg = jnp.concatenate([jnp.zeros((B, S, 1), jnp.float32), asrt], axis=2)
    v_seg = jnp.concatenate([old_v[:, :, None], ts < 1.5], axis=2)
    s_seg = jnp.broadcast_to(seg[:, :, None], (B, S, K + 1))
    L = S * (K + 1)
    t_all = jnp.concatenate([t_seg.reshape(B, L), t1[:, -1:]], axis=1)
    a_all = jnp.concatenate([a_seg.reshape(B, L), jnp.zeros((B, 1), jnp.float32)], axis=1)
    v_all = jnp.concatenate([v_seg.reshape(B, L), n1 == K], axis=1)
    s_all = jnp.concatenate([s_seg.reshape(B, L), jnp.full((B, 1), S, jnp.int32)], axis=1)
    pos = jnp.cumsum(v_all.astype(jnp.int32), axis=1) - 1
    keep = v_all & (pos < MK)
    idx_out = jnp.where(keep, pos, 0)
    b_idx = jnp.arange(B)[:, None]
    t2 = jnp.zeros((B, MK), jnp.float32).at[b_idx, idx_out].add(jnp.where(keep, t_all, 0.0))
    a2s = jnp.zeros((B, MK), jnp.float32).at[b_idx, idx_out].add(jnp.where(keep, a_all, 0.0))
    s2 = jnp.zeros((B, MK), jnp.int32).at[b_idx, idx_out].add(jnp.where(keep, s_all, 0))
    n2 = jnp.minimum(jnp.sum(v_all.astype(jnp.int32), axis=1), MK)
    slot = jax.lax.broadcasted_iota(jnp.int32, (B, MK), 1)
    t2 = jnp.where(slot < n2[:, None], t2, 1.0)
    zl = jnp.take_along_axis(z2, s2[:, :, None], axis=1)
    zr2 = jnp.take_along_axis(z2, jnp.minimum(s2 + 1, K - 1)[:, :, None], axis=1)
    zk = zl + a2s[:, :, None] * (zr2 - zl)
    h2 = jnp.where(slot[:, :, None] < n2[:, None, None], jnp.maximum(zk, 0.0), 0.0)
    return t2, n2, h2


def _sc2_body(t1_hbm, m1_hbm, tc_hbm, aa_hbm, z2_hbm,
              t2_hbm, m2_hbm, h2_hbm,
              t1_v, m1_v, tc_v, aa_v, z2_v, ot_v, os_v, oa_v, h2_v, m2_v):
    wid = lax.axis_index("c") * _NS + lax.axis_index("s")
    iota = lax.broadcasted_iota(jnp.int32, (_NL,), 0)
    lane0 = iota == 0
    onef = jnp.full((_NL,), 1.0, jnp.float32)
    zerof = jnp.zeros((_NL,), jnp.float32)
    zeroi = jnp.zeros((_NL,), jnp.int32)
    S = MK - 1

    for r in range(_RPW):
        ray = wid * _RPW + r
        pltpu.sync_copy(t1_hbm.at[ray], t1_v)
        pltpu.sync_copy(m1_hbm.at[ray], m1_v)
        pltpu.sync_copy(tc_hbm.at[ray], tc_v)
        pltpu.sync_copy(aa_hbm.at[ray], aa_v)
        pltpu.sync_copy(z2_hbm.at[ray], z2_v)
        n1 = m1_v[pl.ds(0, _NL)][0]
        for c in range(4):
            ot_v[pl.ds(c * _NL, _NL)] = onef
            os_v[pl.ds(c * _NL, _NL)] = zeroi
            oa_v[pl.ds(c * _NL, _NL)] = zerof

        def seg_body(s, cur):
            pred = (s <= n1 - 2) & (cur < MK)

            @pl.when(pred)
            def _():
                tv = plsc.load_gather(t1_v, [zeroi + s])
                plsc.store_scatter(ot_v, [zeroi + cur], tv, mask=lane0)
                plsc.store_scatter(os_v, [zeroi + cur], zeroi + s, mask=lane0)

            cur1 = jnp.where(pred, cur + 1, cur)
            ks, ms, cnt = [], [], jnp.int32(0)
            for c in range(4):
                k = tc_v[s, pl.ds(c * _NL, _NL)]
                ks.append(k)
                m = k < 1.5
                ms.append(m)
                cnt = cnt + jnp.sum(m.astype(jnp.int32))
            take = jnp.maximum(jnp.minimum(cnt, MK - cur1), 0)
            do = pred & (cnt > 0) & (cur1 < MK)

            @pl.when(do)
            def _():
                p0 = plsc.sort_key_val(ks[0], iota)
                p1 = plsc.sort_key_val(ks[1], iota + _NL)
                p2 = plsc.sort_key_val(ks[2], iota + 2 * _NL)
                p3 = plsc.sort_key_val(ks[3], iota + 3 * _NL)
                q = _merge2kv(p0[0], p0[1], p1[0], p1[1])
                w = _merge2kv(p2[0], p2[1], p3[0], p3[1])
                f = _merge4kv(*q, *w)
                for i in range(4):
                    pos = iota + i * _NL
                    m = pos < take
                    idx = cur1 + pos
                    plsc.store_scatter(ot_v, [idx], f[2 * i], mask=m)
                    plsc.store_scatter(os_v, [idx], zeroi + s, mask=m)
                    av = plsc.load_gather(aa_v, [zeroi + s, f[2 * i + 1]],
                                          mask=m)
                    plsc.store_scatter(oa_v, [idx], av, mask=m)

            return jnp.where(do, cur1 + take, cur1)

        cur = lax.fori_loop(0, S, seg_body, jnp.int32(0))
        predf = cur < MK

        @pl.when(predf)
        def _():
            tv = plsc.load_gather(t1_v, [zeroi + (n1 - 1)])
            plsc.store_scatter(ot_v, [zeroi + cur], tv, mask=lane0)
            plsc.store_scatter(os_v, [zeroi + cur], zeroi + (n1 - 1),
                               mask=lane0)

        n2 = jnp.where(predf, cur + 1, cur)
        m2_v[pl.ds(0, _NL)] = jnp.where(lane0, n2, 0)

        def z_body(k, carry):
            s = plsc.load_gather(os_v, [zeroi + k])[0]
            av = plsc.load_gather(oa_v, [zeroi + k])
            rr = jnp.minimum(s + 1, MK - 1)
            vf = (k < n2).astype(jnp.float32)
            for c in range(4):
                zl = z2_v[s, pl.ds(c * _NL, _NL)]
                zr = z2_v[rr, pl.ds(c * _NL, _NL)]
                h = jnp.maximum(zl + av * (zr - zl), 0.0) * vf
                h2_v[k, pl.ds(c * _NL, _NL)] = h
            return carry

        lax.fori_loop(0, MK, z_body, jnp.int32(0))
        pltpu.sync_copy(ot_v, t2_hbm.at[ray])
        pltpu.sync_copy(m2_v, m2_hbm.at[ray])
        pltpu.sync_copy(h2_v, h2_hbm.at[ray])


def _sc2(t1, meta1, t2c, a2, z2):
    B = t1.shape[0]
    S = MK - 1
    H = z2.shape[2]
    mesh = plsc.VectorSubcoreMesh(core_axis_name="c", subcore_axis_name="s")
    f = functools.partial(
        pl.kernel,
        out_type=[
            jax.ShapeDtypeStruct((B, MK), jnp.float32),
            jax.ShapeDtypeStruct((B, 16), jnp.int32),
            jax.ShapeDtypeStruct((B, MK, H), jnp.float32),
        ],
        mesh=mesh,
        scratch_types=[
            pltpu.VMEM((MK,), jnp.float32),
            pltpu.VMEM((16,), jnp.int32),
            pltpu.VMEM((S, H), jnp.float32),
            pltpu.VMEM((S, H), jnp.float32),
            pltpu.VMEM((MK, H), jnp.float32),
            pltpu.VMEM((MK,), jnp.float32),
            pltpu.VMEM((MK,), jnp.int32),
            pltpu.VMEM((MK,), jnp.float32),
            pltpu.VMEM((MK, H), jnp.float32),
            pltpu.VMEM((16,), jnp.int32),
        ],
        compiler_params=pltpu.CompilerParams(needs_layout_passes=False),
    )(_sc2_body)
    t2, meta2, h2 = f(t1, meta1, t2c, a2, z2)
    return t2, meta2[:, :1], h2


def kernel(end_points, start_points, W0, b0, W1, b1, W2, b2):
    a0, be0, t1c = _tc_a(end_points, start_points, W0.T, b0)
    t1, meta1 = _sc1(t1c)
    z2, t2c, a2 = _tc_b(a0, be0, t1, meta1[:, :1], meta1[:, 1:2], W1.T, b1)
    t2, n2, h2 = _sc2(t1, meta1, t2c, a2, z2)
    y, v2 = _tc_c(h2, W2.T, b2, n2)
    return t2, v2.astype(bool), y
